# baseline (speedup 1.0000x reference)
import functools

import jax
import jax.numpy as jnp
from jax import lax
from jax.experimental import pallas as pl
from jax.experimental.pallas import tpu as pltpu
from jax.experimental.pallas import tpu_sc as plsc

N = 10000
E = 320000
IN_DIM = 128
HID = 128
CLS = 32

_PREC = lax.Precision.HIGHEST

_NW = 32
_R = 320
_NPAD = _NW * _R
_C = 8000
_G = 128
_F = IN_DIM // 16
_WL = 8320


def _pool_body(m_hbm, src_hbm, dst_hbm, out_hbm,
               ds0, ss0, ds1, ss1, wl_src, wl_dstl, rows0, rows1, pooled,
               sS0, sS1, sG0, sG1):
    i32 = jnp.int32
    wid = lax.axis_index("s") * i32(2) + lax.axis_index("c")
    lo = wid * i32(_R)
    zi = jnp.zeros((16,), jnp.int32)
    zf = jnp.zeros((16,), jnp.float32)
    ri = jnp.full((16,), _R, jnp.int32)
    nchunk = E // _C

    def _z_pooled(r, _):
        for f in range(_F):
            pooled[r, pl.ds(f * 16, 16)] = zf
        return 0
    lax.fori_loop(i32(0), i32(_R + 1), _z_pooled, 0)

    def _z_wl(i, _):
        wl_src[pl.ds(i * i32(16), 16)] = zi
        wl_dstl[pl.ds(i * i32(16), 16)] = ri
        return 0
    lax.fori_loop(i32(0), i32(_WL // 16), _z_wl, 0)

    def _stage_start(ch, dbuf, sbuf, sem):
        base = jnp.minimum(ch, i32(nchunk - 1)) * i32(_C)
        pltpu.make_async_copy(dst_hbm.at[pl.ds(base, _C)], dbuf, sem).start()
        pltpu.make_async_copy(src_hbm.at[pl.ds(base, _C)], sbuf, sem).start()

    def _stage_wait(dbuf, sbuf, sem):
        pltpu.make_async_copy(dst_hbm.at[pl.ds(0, _C)], dbuf, sem).wait()
        pltpu.make_async_copy(src_hbm.at[pl.ds(0, _C)], sbuf, sem).wait()

    def _gather_start(b, rbuf, sem):
        idx = wl_src.at[pl.ds(b * i32(_G), _G)]
        pltpu.make_async_copy(m_hbm.at[idx], rbuf, sem).start()

    def _gather_wait(b, rbuf, sem):
        idx = wl_src.at[pl.ds(b * i32(_G), _G)]
        pltpu.make_async_copy(m_hbm.at[idx], rbuf, sem).wait()

    def _process(b, rbuf):
        bG = b * i32(_G)

        def _edge16(q, _):
            q16 = q * i32(16)
            dls = wl_dstl[pl.ds(bG + q16, 16)]
            for l in range(16):
                dl = dls[l]
                r = q16 + i32(l)
                for f in range(_F):
                    sl = pl.ds(f * 16, 16)
                    pooled[dl, sl] = jnp.maximum(pooled[dl, sl], rbuf[r, sl])
            return 0
        lax.fori_loop(i32(0), i32(_G // 16), _edge16, 0)

    def _scan(dbuf, sbuf):
        def _scan_v(v, cnt):
            v16 = v * i32(16)
            d = dbuf[pl.ds(v16, 16)]
            msk = (d >= lo) & (d < lo + i32(_R))
            pfx = plsc.cumsum(jnp.where(msk, i32(1), i32(0)))
            idx = cnt + pfx - i32(1)
            s = sbuf[pl.ds(v16, 16)]
            plsc.store_scatter(wl_src, [idx], s, mask=msk)
            plsc.store_scatter(wl_dstl, [idx], d - lo, mask=msk)
            return cnt + plsc.all_reduce_population_count(msk)

        cnt_v = lax.fori_loop(i32(0), i32(_C // 16), _scan_v,
                              jnp.zeros((16,), jnp.int32))
        return cnt_v[0]

    def _chunk(dbuf, sbuf, ssem):
        _stage_wait(dbuf, sbuf, ssem)
        cnt = _scan(dbuf, sbuf)
        nb2 = (cnt + i32(2 * _G - 1)) // i32(2 * _G)
        _gather_start(i32(0), rows0, sG0)

        def _pair(p, _):
            p2 = p * i32(2)
            _gather_start(p2 + i32(1), rows1, sG1)
            _gather_wait(p2, rows0, sG0)
            _process(p2, rows0)
            _gather_start(p2 + i32(2), rows0, sG0)
            _gather_wait(p2 + i32(1), rows1, sG1)
            _process(p2 + i32(1), rows1)
            return 0

        lax.fori_loop(i32(0), nb2, _pair, 0)
        _gather_wait(i32(0), rows0, sG0)

    _stage_start(i32(0), ds0, ss0, sS0)

    def _cpair(cp, _):
        c2 = cp * i32(2)
        _stage_start(c2 + i32(1), ds1, ss1, sS1)
        _chunk(ds0, ss0, sS0)
        _stage_start(c2 + i32(2), ds0, ss0, sS0)
        _chunk(ds1, ss1, sS1)
        return 0

    lax.fori_loop(i32(0), i32(nchunk // 2), _cpair, 0)
    _stage_wait(ds0, ss0, sS0)
    pltpu.sync_copy(pooled.at[pl.ds(0, _R)], out_hbm.at[pl.ds(lo, _R)])


@functools.partial(jax.jit, static_argnames=())
def _segment_max_sc(m, src, dst):
    mesh = plsc.VectorSubcoreMesh(core_axis_name="c", subcore_axis_name="s")
    call = pl.kernel(
        _pool_body,
        out_type=jax.ShapeDtypeStruct((_NPAD, IN_DIM), jnp.float32),
        mesh=mesh,
        scratch_types=[
            pltpu.VMEM((_C,), jnp.int32),
            pltpu.VMEM((_C,), jnp.int32),
            pltpu.VMEM((_C,), jnp.int32),
            pltpu.VMEM((_C,), jnp.int32),
            pltpu.VMEM((_WL,), jnp.int32),
            pltpu.VMEM((_WL,), jnp.int32),
            pltpu.VMEM((_G, IN_DIM), jnp.float32),
            pltpu.VMEM((_G, IN_DIM), jnp.float32),
            pltpu.VMEM((_R + 1, IN_DIM), jnp.float32),
            pltpu.SemaphoreType.DMA,
            pltpu.SemaphoreType.DMA,
            pltpu.SemaphoreType.DMA,
            pltpu.SemaphoreType.DMA,
        ],
        compiler_params=pltpu.CompilerParams(needs_layout_passes=False),
    )
    return call(m, src, dst)[:N]


def _pre_body(h_ref, wp_ref, bp_ref, ws_ref, m_ref, hs_ref):
    h = h_ref[...]
    m_ref[...] = jnp.maximum(
        jnp.dot(h, wp_ref[...].T, precision=_PREC) + bp_ref[...], 0.0)
    hs_ref[...] = jnp.dot(h, ws_ref[...].T, precision=_PREC)


def _mid_body(hs_ref, p_ref, wn_ref, b_ref, wp2_ref, bp2_ref, ws2_ref,
              m2_ref, hs2_ref):
    x = hs_ref[...] + jnp.dot(p_ref[...], wn_ref[...].T, precision=_PREC) + b_ref[...]
    h1 = jnp.where(x > 0, x, jnp.exp(jnp.minimum(x, 0.0)) - 1.0)
    m2_ref[...] = jnp.maximum(
        jnp.dot(h1, wp2_ref[...].T, precision=_PREC) + bp2_ref[...], 0.0)
    hs2_ref[...] = jnp.dot(h1, ws2_ref[...].T, precision=_PREC)


def _post_body(hs2_ref, p2_ref, wn2_ref, b2_ref, out_ref):
    logits = (hs2_ref[...] + jnp.dot(p2_ref[...], wn2_ref[...].T, precision=_PREC)
              + b2_ref[...])
    out_ref[...] = jnp.mean(logits, axis=1, keepdims=True)


def _segment_max(m, src, dst):
    return _segment_max_sc(m, src, dst)


def kernel(h, edge_index, Wp1, bp1, Ws1, Wn1, b1, Wp2, bp2, Ws2, Wn2, b2):
    src = edge_index[0].astype(jnp.int32)
    dst = edge_index[1].astype(jnp.int32)

    m1, hs1 = pl.pallas_call(
        _pre_body,
        out_shape=[jax.ShapeDtypeStruct((N, IN_DIM), jnp.float32),
                   jax.ShapeDtypeStruct((N, HID), jnp.float32)],
    )(h, Wp1, bp1, Ws1)

    pooled1 = _segment_max(m1, src, dst)

    m2, hs2 = pl.pallas_call(
        _mid_body,
        out_shape=[jax.ShapeDtypeStruct((N, HID), jnp.float32),
                   jax.ShapeDtypeStruct((N, CLS), jnp.float32)],
    )(hs1, pooled1, Wn1, b1, Wp2, bp2, Ws2)

    pooled2 = _segment_max(m2, src, dst)

    out = pl.pallas_call(
        _post_body,
        out_shape=jax.ShapeDtypeStruct((N, 1), jnp.float32),
    )(hs2, pooled2, Wn2, b2)
    return out.reshape(N)


# speedup vs baseline: 1.0007x; 1.0007x over previous
import functools

import jax
import jax.numpy as jnp
from jax import lax
from jax.experimental import pallas as pl
from jax.experimental.pallas import tpu as pltpu
from jax.experimental.pallas import tpu_sc as plsc

N = 10000
E = 320000
IN_DIM = 128
HID = 128
CLS = 32

_PREC = lax.Precision.HIGHEST

_NW = 32
_R = 320
_NPAD = _NW * _R
_C = 8000
_G = 128
_F = IN_DIM // 16
_WL = 8320


def _pool_body(m_hbm, src_hbm, dst_hbm, out_hbm,
               ds0, ss0, ds1, ss1, wl_src, wl_dstl, idx0, idx1,
               rows0, rows1, pooled, sS0, sS1, sG0, sG1):
    i32 = jnp.int32
    wid = lax.axis_index("s") * i32(2) + lax.axis_index("c")
    lo = wid * i32(_R)
    zi = jnp.zeros((16,), jnp.int32)
    zf = jnp.zeros((16,), jnp.float32)
    ri = jnp.full((16,), _R, jnp.int32)
    nchunk = E // _C

    def _z_pooled(r, _):
        for f in range(_F):
            pooled[r, pl.ds(f * 16, 16)] = zf
        return 0
    lax.fori_loop(i32(0), i32(_R + 1), _z_pooled, 0)

    def _z_wl(i, _):
        wl_src[pl.ds(i * i32(16), 16)] = zi
        wl_dstl[pl.ds(i * i32(16), 16)] = ri
        return 0
    lax.fori_loop(i32(0), i32(_WL // 16), _z_wl, 0)

    def _stage_start(ch, dbuf, sbuf, sem):
        base = jnp.minimum(ch, i32(nchunk - 1)) * i32(_C)
        pltpu.make_async_copy(dst_hbm.at[pl.ds(base, _C)], dbuf, sem).start()
        pltpu.make_async_copy(src_hbm.at[pl.ds(base, _C)], sbuf, sem).start()

    def _stage_wait(dbuf, sbuf, sem):
        pltpu.make_async_copy(dst_hbm.at[pl.ds(0, _C)], dbuf, sem).wait()
        pltpu.make_async_copy(src_hbm.at[pl.ds(0, _C)], sbuf, sem).wait()

    def _gather_start(b, ibuf, rbuf, sem):
        bG = b * i32(_G)
        for i in range(_G // 16):
            ibuf[pl.ds(i * 16, 16)] = wl_src[pl.ds(bG + i32(i * 16), 16)]
        pltpu.make_async_copy(m_hbm.at[ibuf], rbuf, sem).start()

    def _gather_wait(ibuf, rbuf, sem):
        pltpu.make_async_copy(m_hbm.at[ibuf], rbuf, sem).wait()

    def _process(b, rbuf):
        bG = b * i32(_G)

        def _edge16(q, _):
            q16 = q * i32(16)
            dls = wl_dstl[pl.ds(bG + q16, 16)]
            for l in range(16):
                dl = dls[l]
                r = q16 + i32(l)
                for f in range(_F):
                    sl = pl.ds(f * 16, 16)
                    pooled[dl, sl] = jnp.maximum(pooled[dl, sl], rbuf[r, sl])
            return 0
        lax.fori_loop(i32(0), i32(_G // 16), _edge16, 0)

    def _scan(dbuf, sbuf):
        def _scan_v(v, cnt):
            v16 = v * i32(16)
            d = dbuf[pl.ds(v16, 16)]
            msk = (d >= lo) & (d < lo + i32(_R))
            pfx = plsc.cumsum(jnp.where(msk, i32(1), i32(0)))
            idx = cnt + pfx - i32(1)
            s = sbuf[pl.ds(v16, 16)]
            plsc.store_scatter(wl_src, [idx], s, mask=msk)
            plsc.store_scatter(wl_dstl, [idx], d - lo, mask=msk)
            return cnt + plsc.all_reduce_population_count(msk)

        cnt_v = lax.fori_loop(i32(0), i32(_C // 16), _scan_v,
                              jnp.zeros((16,), jnp.int32))
        return cnt_v[0]

    def _chunk(dbuf, sbuf, ssem):
        _stage_wait(dbuf, sbuf, ssem)
        with jax.named_scope("edge_scan"):
            cnt = _scan(dbuf, sbuf)
        nb2 = (cnt + i32(2 * _G - 1)) // i32(2 * _G)

        with jax.named_scope("pool_update"):
            _gather_start(i32(0), idx0, rows0, sG0)

            def _pair(p, _):
                p2 = p * i32(2)
                _gather_start(p2 + i32(1), idx1, rows1, sG1)
                _gather_wait(idx0, rows0, sG0)
                _process(p2, rows0)
                _gather_start(p2 + i32(2), idx0, rows0, sG0)
                _gather_wait(idx1, rows1, sG1)
                _process(p2 + i32(1), rows1)
                return 0

            lax.fori_loop(i32(0), nb2, _pair, 0)
            _gather_wait(idx0, rows0, sG0)

    _stage_start(i32(0), ds0, ss0, sS0)

    def _cpair(cp, _):
        c2 = cp * i32(2)
        _stage_start(c2 + i32(1), ds1, ss1, sS1)
        _chunk(ds0, ss0, sS0)
        _stage_start(c2 + i32(2), ds0, ss0, sS0)
        _chunk(ds1, ss1, sS1)
        return 0

    lax.fori_loop(i32(0), i32(nchunk // 2), _cpair, 0)
    _stage_wait(ds0, ss0, sS0)
    pltpu.sync_copy(pooled.at[pl.ds(0, _R)], out_hbm.at[pl.ds(lo, _R)])


@functools.partial(jax.jit, static_argnames=())
def _segment_max_sc(m, src, dst):
    mesh = plsc.VectorSubcoreMesh(core_axis_name="c", subcore_axis_name="s")
    call = pl.kernel(
        _pool_body,
        out_type=jax.ShapeDtypeStruct((_NPAD, IN_DIM), jnp.float32),
        mesh=mesh,
        scratch_types=[
            pltpu.VMEM((_C,), jnp.int32),
            pltpu.VMEM((_C,), jnp.int32),
            pltpu.VMEM((_C,), jnp.int32),
            pltpu.VMEM((_C,), jnp.int32),
            pltpu.VMEM((_WL,), jnp.int32),
            pltpu.VMEM((_WL,), jnp.int32),
            pltpu.VMEM((_G,), jnp.int32),
            pltpu.VMEM((_G,), jnp.int32),
            pltpu.VMEM((_G, IN_DIM), jnp.float32),
            pltpu.VMEM((_G, IN_DIM), jnp.float32),
            pltpu.VMEM((_R + 1, IN_DIM), jnp.float32),
            pltpu.SemaphoreType.DMA,
            pltpu.SemaphoreType.DMA,
            pltpu.SemaphoreType.DMA,
            pltpu.SemaphoreType.DMA,
        ],
        compiler_params=pltpu.CompilerParams(needs_layout_passes=False),
    )
    return call(m, src, dst)[:N]


def _pre_body(h_ref, wp_ref, bp_ref, ws_ref, m_ref, hs_ref):
    h = h_ref[...]
    m_ref[...] = jnp.maximum(
        jnp.dot(h, wp_ref[...].T, precision=_PREC) + bp_ref[...], 0.0)
    hs_ref[...] = jnp.dot(h, ws_ref[...].T, precision=_PREC)


def _mid_body(hs_ref, p_ref, wn_ref, b_ref, wp2_ref, bp2_ref, ws2_ref,
              m2_ref, hs2_ref):
    x = hs_ref[...] + jnp.dot(p_ref[...], wn_ref[...].T, precision=_PREC) + b_ref[...]
    h1 = jnp.where(x > 0, x, jnp.exp(jnp.minimum(x, 0.0)) - 1.0)
    m2_ref[...] = jnp.maximum(
        jnp.dot(h1, wp2_ref[...].T, precision=_PREC) + bp2_ref[...], 0.0)
    hs2_ref[...] = jnp.dot(h1, ws2_ref[...].T, precision=_PREC)


def _post_body(hs2_ref, p2_ref, wn2_ref, b2_ref, out_ref):
    logits = (hs2_ref[...] + jnp.dot(p2_ref[...], wn2_ref[...].T, precision=_PREC)
              + b2_ref[...])
    out_ref[...] = jnp.mean(logits, axis=1, keepdims=True)


def _segment_max(m, src, dst):
    return _segment_max_sc(m, src, dst)


def kernel(h, edge_index, Wp1, bp1, Ws1, Wn1, b1, Wp2, bp2, Ws2, Wn2, b2):
    src = edge_index[0].astype(jnp.int32)
    dst = edge_index[1].astype(jnp.int32)

    m1, hs1 = pl.pallas_call(
        _pre_body,
        out_shape=[jax.ShapeDtypeStruct((N, IN_DIM), jnp.float32),
                   jax.ShapeDtypeStruct((N, HID), jnp.float32)],
    )(h, Wp1, bp1, Ws1)

    pooled1 = _segment_max(m1, src, dst)

    m2, hs2 = pl.pallas_call(
        _mid_body,
        out_shape=[jax.ShapeDtypeStruct((N, HID), jnp.float32),
                   jax.ShapeDtypeStruct((N, CLS), jnp.float32)],
    )(hs1, pooled1, Wn1, b1, Wp2, bp2, Ws2)

    pooled2 = _segment_max(m2, src, dst)

    out = pl.pallas_call(
        _post_body,
        out_shape=jax.ShapeDtypeStruct((N, 1), jnp.float32),
    )(hs2, pooled2, Wn2, b2)
    return out.reshape(N)


# speedup vs baseline: 11.6118x; 11.6037x over previous
import functools

import jax
import jax.numpy as jnp
from jax import lax
from jax.experimental import pallas as pl
from jax.experimental.pallas import tpu as pltpu
from jax.experimental.pallas import tpu_sc as plsc

N = 10000
E = 320000
IN_DIM = 128
HID = 128
CLS = 32

_PREC = lax.Precision.HIGHEST

_NW = 32
_R = 320
_NPAD = _NW * _R
_C = 8000
_G = 128
_F = IN_DIM // 16
_WL = 8320


def _pool_body(m_hbm, src_hbm, dst_hbm, out_hbm,
               ds0, ss0, ds1, ss1, wl_src, wl_dstl, idx0, idx1,
               rows0, rows1, pooled, sS0, sS1, sG0, sG1):
    i32 = jnp.int32
    wid = lax.axis_index("s") * i32(2) + lax.axis_index("c")
    lo = wid * i32(_R)
    zi = jnp.zeros((16,), jnp.int32)
    zf = jnp.zeros((16,), jnp.float32)
    ri = jnp.full((16,), _R, jnp.int32)
    nchunk = E // _C

    def _z_pooled(r, _):
        for f in range(_F):
            pooled[r, pl.ds(f * 16, 16)] = zf
        return 0
    lax.fori_loop(i32(0), i32(_R + 1), _z_pooled, 0)

    lanes = lax.iota(jnp.int32, 16)

    def _z_wl(i, _):
        spread = (i * i32(16) + lanes + lo) & i32(0x1FFF)
        wl_src[pl.ds(i * i32(16), 16)] = spread
        wl_dstl[pl.ds(i * i32(16), 16)] = ri
        return 0
    lax.fori_loop(i32(0), i32(_WL // 16), _z_wl, 0)

    def _gather_start(b, ibuf, rbuf, sem):
        bG = b * i32(_G)
        for i in range(_G // 16):
            ibuf[pl.ds(i * 16, 16)] = wl_src[pl.ds(bG + i32(i * 16), 16)]
        pltpu.make_async_copy(m_hbm.at[ibuf], rbuf, sem).start()

    def _gather_wait(ibuf, rbuf, sem):
        pltpu.make_async_copy(m_hbm.at[ibuf], rbuf, sem).wait()

    def _process(b, rbuf):
        bG = b * i32(_G)

        def _edge16(q, _):
            q16 = q * i32(16)
            dls = wl_dstl[pl.ds(bG + q16, 16)]
            for l in range(16):
                dl = dls[l]
                r = q16 + i32(l)
                for f in range(_F):
                    sl = pl.ds(f * 16, 16)
                    pooled[dl, sl] = jnp.maximum(pooled[dl, sl], rbuf[r, sl])
            return 0
        lax.fori_loop(i32(0), i32(_G // 16), _edge16, 0)

    def _scan(dbuf, sbuf):
        def _scan_v(v, cnt):
            v16 = v * i32(16)
            d = dbuf[pl.ds(v16, 16)]
            msk = (d >= lo) & (d < lo + i32(_R))
            pfx = plsc.cumsum(jnp.where(msk, i32(1), i32(0)))
            idx = cnt + pfx - i32(1)
            s = sbuf[pl.ds(v16, 16)]
            plsc.store_scatter(wl_src, [idx], s, mask=msk)
            plsc.store_scatter(wl_dstl, [idx], d - lo, mask=msk)
            return cnt + plsc.all_reduce_population_count(msk)

        cnt_v = lax.fori_loop(i32(0), i32(_C // 16), _scan_v,
                              jnp.zeros((16,), jnp.int32))
        return cnt_v[0]

    def _chunk(ch, _):
        base = ch * i32(_C)
        pltpu.sync_copy(dst_hbm.at[pl.ds(base, _C)], ds0)
        pltpu.sync_copy(src_hbm.at[pl.ds(base, _C)], ss0)
        cnt = _scan(ds0, ss0)
        nb = (cnt + i32(_G - 1)) // i32(_G)

        def _batch(b, _):
            _gather_start(b, idx0, rows0, sG0)
            _gather_wait(idx0, rows0, sG0)
            _process(b, rows0)
            return 0

        lax.fori_loop(i32(0), nb, _batch, 0)
        return 0

    lax.fori_loop(i32(0), i32(nchunk), _chunk, 0)
    pltpu.sync_copy(pooled.at[pl.ds(0, _R)], out_hbm.at[pl.ds(lo, _R)])


@functools.partial(jax.jit, static_argnames=())
def _segment_max_sc(m, src, dst):
    mesh = plsc.VectorSubcoreMesh(core_axis_name="c", subcore_axis_name="s")
    call = pl.kernel(
        _pool_body,
        out_type=jax.ShapeDtypeStruct((_NPAD, IN_DIM), jnp.float32),
        mesh=mesh,
        scratch_types=[
            pltpu.VMEM((_C,), jnp.int32),
            pltpu.VMEM((_C,), jnp.int32),
            pltpu.VMEM((_C,), jnp.int32),
            pltpu.VMEM((_C,), jnp.int32),
            pltpu.VMEM((_WL,), jnp.int32),
            pltpu.VMEM((_WL,), jnp.int32),
            pltpu.VMEM((_G,), jnp.int32),
            pltpu.VMEM((_G,), jnp.int32),
            pltpu.VMEM((_G, IN_DIM), jnp.float32),
            pltpu.VMEM((_G, IN_DIM), jnp.float32),
            pltpu.VMEM((_R + 1, IN_DIM), jnp.float32),
            pltpu.SemaphoreType.DMA,
            pltpu.SemaphoreType.DMA,
            pltpu.SemaphoreType.DMA,
            pltpu.SemaphoreType.DMA,
        ],
        compiler_params=pltpu.CompilerParams(needs_layout_passes=False),
    )
    return call(m, src, dst)[:N]


def _pre_body(h_ref, wp_ref, bp_ref, ws_ref, m_ref, hs_ref):
    h = h_ref[...]
    m_ref[...] = jnp.maximum(
        jnp.dot(h, wp_ref[...].T, precision=_PREC) + bp_ref[...], 0.0)
    hs_ref[...] = jnp.dot(h, ws_ref[...].T, precision=_PREC)


def _mid_body(hs_ref, p_ref, wn_ref, b_ref, wp2_ref, bp2_ref, ws2_ref,
              m2_ref, hs2_ref):
    x = hs_ref[...] + jnp.dot(p_ref[...], wn_ref[...].T, precision=_PREC) + b_ref[...]
    h1 = jnp.where(x > 0, x, jnp.exp(jnp.minimum(x, 0.0)) - 1.0)
    m2_ref[...] = jnp.maximum(
        jnp.dot(h1, wp2_ref[...].T, precision=_PREC) + bp2_ref[...], 0.0)
    hs2_ref[...] = jnp.dot(h1, ws2_ref[...].T, precision=_PREC)


def _post_body(hs2_ref, p2_ref, wn2_ref, b2_ref, out_ref):
    logits = (hs2_ref[...] + jnp.dot(p2_ref[...], wn2_ref[...].T, precision=_PREC)
              + b2_ref[...])
    out_ref[...] = jnp.mean(logits, axis=1, keepdims=True)


def _segment_max(m, src, dst):
    return _segment_max_sc(m, src, dst)


def kernel(h, edge_index, Wp1, bp1, Ws1, Wn1, b1, Wp2, bp2, Ws2, Wn2, b2):
    src = edge_index[0].astype(jnp.int32)
    dst = edge_index[1].astype(jnp.int32)

    m1, hs1 = pl.pallas_call(
        _pre_body,
        out_shape=[jax.ShapeDtypeStruct((N, IN_DIM), jnp.float32),
                   jax.ShapeDtypeStruct((N, HID), jnp.float32)],
    )(h, Wp1, bp1, Ws1)

    pooled1 = _segment_max(m1, src, dst)

    m2, hs2 = pl.pallas_call(
        _mid_body,
        out_shape=[jax.ShapeDtypeStruct((N, HID), jnp.float32),
                   jax.ShapeDtypeStruct((N, CLS), jnp.float32)],
    )(hs1, pooled1, Wn1, b1, Wp2, bp2, Ws2)

    pooled2 = _segment_max(m2, src, dst)

    out = pl.pallas_call(
        _post_body,
        out_shape=jax.ShapeDtypeStruct((N, 1), jnp.float32),
    )(hs2, pooled2, Wn2, b2)
    return out.reshape(N)


# speedup vs baseline: 12.3595x; 1.0644x over previous
import functools

import jax
import jax.numpy as jnp
from jax import lax
from jax.experimental import pallas as pl
from jax.experimental.pallas import tpu as pltpu
from jax.experimental.pallas import tpu_sc as plsc

N = 10000
E = 320000
IN_DIM = 128
HID = 128
CLS = 32

_PREC = lax.Precision.HIGHEST

_NW = 32
_R = 320
_NPAD = _NW * _R
_C = 8000
_G = 128
_F = IN_DIM // 16
_WL = 8320


def _pool_body(m_hbm, src_hbm, dst_hbm, out_hbm,
               ds0, ss0, ds1, ss1, wl_src, wl_dstl, idx0, idx1,
               rows0, rows1, pooled, sS0, sS1, sG0, sG1):
    i32 = jnp.int32
    wid = lax.axis_index("s") * i32(2) + lax.axis_index("c")
    lo = wid * i32(_R)
    zi = jnp.zeros((16,), jnp.int32)
    zf = jnp.zeros((16,), jnp.float32)
    ri = jnp.full((16,), _R, jnp.int32)
    nchunk = E // _C

    def _z_pooled(r, _):
        for f in range(_F):
            pooled[r, pl.ds(f * 16, 16)] = zf
        return 0
    lax.fori_loop(i32(0), i32(_R + 1), _z_pooled, 0)

    lanes = lax.iota(jnp.int32, 16)

    def _z_wl(i, _):
        spread = (i * i32(16) + lanes + lo) & i32(0x1FFF)
        wl_src[pl.ds(i * i32(16), 16)] = spread
        wl_dstl[pl.ds(i * i32(16), 16)] = ri
        return 0
    lax.fori_loop(i32(0), i32(_WL // 16), _z_wl, 0)

    def _gather_start(b, ibuf, rbuf, sem):
        bG = b * i32(_G)
        for i in range(_G // 16):
            ibuf[pl.ds(i * 16, 16)] = wl_src[pl.ds(bG + i32(i * 16), 16)]
        pltpu.make_async_copy(m_hbm.at[ibuf], rbuf, sem).start()

    def _gather_wait(ibuf, rbuf, sem):
        pltpu.make_async_copy(m_hbm.at[ibuf], rbuf, sem).wait()

    def _process(b, rbuf):
        bG = b * i32(_G)

        def _edge16(q, _):
            q16 = q * i32(16)
            dls = wl_dstl[pl.ds(bG + q16, 16)]
            for l in range(16):
                dl = dls[l]
                r = q16 + i32(l)
                for f in range(_F):
                    sl = pl.ds(f * 16, 16)
                    pooled[dl, sl] = jnp.maximum(pooled[dl, sl], rbuf[r, sl])
            return 0
        lax.fori_loop(i32(0), i32(_G // 16), _edge16, 0)

    def _scan(dbuf, sbuf):
        def _scan_v(v, cnt):
            v16 = v * i32(16)
            d = dbuf[pl.ds(v16, 16)]
            msk = (d >= lo) & (d < lo + i32(_R))
            pfx = plsc.cumsum(jnp.where(msk, i32(1), i32(0)))
            idx = cnt + pfx - i32(1)
            s = sbuf[pl.ds(v16, 16)]
            plsc.store_scatter(wl_src, [idx], s, mask=msk)
            plsc.store_scatter(wl_dstl, [idx], d - lo, mask=msk)
            return cnt + plsc.all_reduce_population_count(msk)

        cnt_v = lax.fori_loop(i32(0), i32(_C // 16), _scan_v,
                              jnp.zeros((16,), jnp.int32))
        return cnt_v[0]

    def _chunk(ch, _):
        base = ch * i32(_C)
        pltpu.sync_copy(dst_hbm.at[pl.ds(base, _C)], ds0)
        pltpu.sync_copy(src_hbm.at[pl.ds(base, _C)], ss0)
        cnt = _scan(ds0, ss0)
        nb = (cnt + i32(_G - 1)) // i32(_G)
        nb2 = (nb + i32(1)) // i32(2)

        @pl.when(nb > i32(0))
        def _():
            _gather_start(i32(0), idx0, rows0, sG0)

        def _pair(p, _):
            p2 = p * i32(2)

            @pl.when(p2 + i32(1) < nb)
            def _():
                _gather_start(p2 + i32(1), idx1, rows1, sG1)

            _gather_wait(idx0, rows0, sG0)
            _process(p2, rows0)

            @pl.when(p2 + i32(2) < nb)
            def _():
                _gather_start(p2 + i32(2), idx0, rows0, sG0)

            @pl.when(p2 + i32(1) < nb)
            def _():
                _gather_wait(idx1, rows1, sG1)
                _process(p2 + i32(1), rows1)
            return 0

        lax.fori_loop(i32(0), nb2, _pair, 0)
        return 0

    lax.fori_loop(i32(0), i32(nchunk), _chunk, 0)
    pltpu.sync_copy(pooled.at[pl.ds(0, _R)], out_hbm.at[pl.ds(lo, _R)])


@functools.partial(jax.jit, static_argnames=())
def _segment_max_sc(m, src, dst):
    mesh = plsc.VectorSubcoreMesh(core_axis_name="c", subcore_axis_name="s")
    call = pl.kernel(
        _pool_body,
        out_type=jax.ShapeDtypeStruct((_NPAD, IN_DIM), jnp.float32),
        mesh=mesh,
        scratch_types=[
            pltpu.VMEM((_C,), jnp.int32),
            pltpu.VMEM((_C,), jnp.int32),
            pltpu.VMEM((_C,), jnp.int32),
            pltpu.VMEM((_C,), jnp.int32),
            pltpu.VMEM((_WL,), jnp.int32),
            pltpu.VMEM((_WL,), jnp.int32),
            pltpu.VMEM((_G,), jnp.int32),
            pltpu.VMEM((_G,), jnp.int32),
            pltpu.VMEM((_G, IN_DIM), jnp.float32),
            pltpu.VMEM((_G, IN_DIM), jnp.float32),
            pltpu.VMEM((_R + 1, IN_DIM), jnp.float32),
            pltpu.SemaphoreType.DMA,
            pltpu.SemaphoreType.DMA,
            pltpu.SemaphoreType.DMA,
            pltpu.SemaphoreType.DMA,
        ],
        compiler_params=pltpu.CompilerParams(needs_layout_passes=False),
    )
    return call(m, src, dst)[:N]


def _pre_body(h_ref, wp_ref, bp_ref, ws_ref, m_ref, hs_ref):
    h = h_ref[...]
    m_ref[...] = jnp.maximum(
        jnp.dot(h, wp_ref[...].T, precision=_PREC) + bp_ref[...], 0.0)
    hs_ref[...] = jnp.dot(h, ws_ref[...].T, precision=_PREC)


def _mid_body(hs_ref, p_ref, wn_ref, b_ref, wp2_ref, bp2_ref, ws2_ref,
              m2_ref, hs2_ref):
    x = hs_ref[...] + jnp.dot(p_ref[...], wn_ref[...].T, precision=_PREC) + b_ref[...]
    h1 = jnp.where(x > 0, x, jnp.exp(jnp.minimum(x, 0.0)) - 1.0)
    m2_ref[...] = jnp.maximum(
        jnp.dot(h1, wp2_ref[...].T, precision=_PREC) + bp2_ref[...], 0.0)
    hs2_ref[...] = jnp.dot(h1, ws2_ref[...].T, precision=_PREC)


def _post_body(hs2_ref, p2_ref, wn2_ref, b2_ref, out_ref):
    logits = (hs2_ref[...] + jnp.dot(p2_ref[...], wn2_ref[...].T, precision=_PREC)
              + b2_ref[...])
    out_ref[...] = jnp.mean(logits, axis=1, keepdims=True)


def _segment_max(m, src, dst):
    return _segment_max_sc(m, src, dst)


def kernel(h, edge_index, Wp1, bp1, Ws1, Wn1, b1, Wp2, bp2, Ws2, Wn2, b2):
    src = edge_index[0].astype(jnp.int32)
    dst = edge_index[1].astype(jnp.int32)

    m1, hs1 = pl.pallas_call(
        _pre_body,
        out_shape=[jax.ShapeDtypeStruct((N, IN_DIM), jnp.float32),
                   jax.ShapeDtypeStruct((N, HID), jnp.float32)],
    )(h, Wp1, bp1, Ws1)

    pooled1 = _segment_max(m1, src, dst)

    m2, hs2 = pl.pallas_call(
        _mid_body,
        out_shape=[jax.ShapeDtypeStruct((N, HID), jnp.float32),
                   jax.ShapeDtypeStruct((N, CLS), jnp.float32)],
    )(hs1, pooled1, Wn1, b1, Wp2, bp2, Ws2)

    pooled2 = _segment_max(m2, src, dst)

    out = pl.pallas_call(
        _post_body,
        out_shape=jax.ShapeDtypeStruct((N, 1), jnp.float32),
    )(hs2, pooled2, Wn2, b2)
    return out.reshape(N)
